# baseline (speedup 1.0000x reference)
import functools

import jax
import jax.numpy as jnp
from jax import lax
from jax.experimental import pallas as pl
from jax.experimental.pallas import tpu as pltpu
from jax.experimental.pallas import tpu_sc as plsc

N = 10000
E = 160000
NP = 10112
ROWS = 1000
GRID = N // ROWS
NTILES = 32
EPT = E // NTILES
EB = 96
K = 1
WAVES = 28
NBLK = 2 * K * WAVES
F32 = jnp.float32


def _leaky(v):
    return jnp.where(v >= 0, v, 0.01 * v)


def _dot(a, b):
    return jnp.dot(a, b, preferred_element_type=F32)


def _sc_spmm_chunk(h, cl3, rw3, vl3, fc):
    mesh = plsc.VectorSubcoreMesh(core_axis_name="c", subcore_axis_name="s")
    rpt = NP // 16
    nlane = fc // 16

    def body(h_hbm, cl_hbm, rw_hbm, vl_hbm, out_hbm,
             cv0, cv1, rv0, rv1, vv0, vv1, gat0, gat1, acc_sh,
             esem, gsem0, gsem1, ssem):
        cid = lax.axis_index("c")
        sid = lax.axis_index("s")
        wid = cid * 16 + sid
        cvs = (cv0, cv1)
        rvs = (rv0, rv1)
        vvs = (vv0, vv1)
        gats = (gat0, gat1)
        gsems = (gsem0, gsem1)

        def zrow(r, _):
            for f in range(nlane):
                gat0[0, r, pl.ds(16 * f, 16)] = jnp.zeros((16,), F32)
            return 0
        lax.fori_loop(0, EB, zrow, 0)
        base = sid * rpt
        nfull = rpt // EB
        rem = rpt - nfull * EB
        zh = []
        for k in range(nfull):
            zh.append(pltpu.async_copy(
                gat0.at[0], acc_sh.at[pl.ds(base + k * EB, EB)], esem))
        zh.append(pltpu.async_copy(
            gat0.at[0, pl.ds(0, rem)],
            acc_sh.at[pl.ds(base + nfull * EB, rem)], esem))
        for hdl in zh:
            hdl.wait()
        plsc.subcore_barrier()

        def scale_slot(vv, gat):
            for k in range(K):
                def grp(g, _):
                    v16 = vv[k, pl.ds(16 * g, 16)]
                    for l in range(16):
                        v = v16[l]
                        r = 16 * g + l
                        for f in range(nlane):
                            sl = pl.ds(16 * f, 16)
                            gat[k, r, sl] = gat[k, r, sl] * v
                    return 0
                lax.fori_loop(0, EB // 16, grp, 0)

        def wave(w, _):
            wb = w * (2 * K)
            eh = []
            for s in range(2):
                for k in range(K):
                    b = wb + s * K + k
                    eh.append(pltpu.async_copy(
                        cl_hbm.at[wid, b], cvs[s].at[k], esem))
                    eh.append(pltpu.async_copy(
                        rw_hbm.at[wid, b], rvs[s].at[k], esem))
                    eh.append(pltpu.async_copy(
                        vl_hbm.at[wid, b], vvs[s].at[k], esem))
            for hdl in eh:
                hdl.wait()
            gh = [[], []]
            for s in range(2):
                for k in range(K):
                    gh[s].append(pltpu.async_copy(
                        h_hbm.at[cvs[s].at[k]], gats[s].at[k], gsems[s]))
            sh = []
            for s in range(2):
                for hdl in gh[s]:
                    hdl.wait()
                scale_slot(vvs[s], gats[s])
                for k in range(K):
                    sh.append(pltpu.async_copy(
                        gats[s].at[k], acc_sh.at[rvs[s].at[k]], ssem,
                        add=True))
            for hdl in sh:
                hdl.wait()
            return 0
        lax.fori_loop(0, WAVES, wave, 0)
        plsc.subcore_barrier()

        wh = []
        for k in range(nfull):
            wh.append(pltpu.async_copy(
                acc_sh.at[pl.ds(base + k * EB, EB)],
                out_hbm.at[cid, pl.ds(base + k * EB, EB)], esem))
        wh.append(pltpu.async_copy(
            acc_sh.at[pl.ds(base + nfull * EB, rem)],
            out_hbm.at[cid, pl.ds(base + nfull * EB, rem)], esem))
        for hdl in wh:
            hdl.wait()

    run = pl.kernel(
        body,
        out_type=jax.ShapeDtypeStruct((2, NP, fc), F32),
        mesh=mesh,
        scratch_types=[
            pltpu.VMEM((K, EB), jnp.int32),
            pltpu.VMEM((K, EB), jnp.int32),
            pltpu.VMEM((K, EB), jnp.int32),
            pltpu.VMEM((K, EB), jnp.int32),
            pltpu.VMEM((K, EB), F32),
            pltpu.VMEM((K, EB), F32),
            pltpu.VMEM((K, EB, fc), F32),
            pltpu.VMEM((K, EB, fc), F32),
            pltpu.VMEM_SHARED((NP, fc), F32),
            pltpu.SemaphoreType.DMA,
            pltpu.SemaphoreType.DMA,
            pltpu.SemaphoreType.DMA,
            pltpu.SemaphoreType.DMA,
        ],
    )
    return run(h, cl3, rw3, vl3)


def _sc_spmm(h_chunks, ed3, fc=128):
    cl, rw, vl = ed3
    return [_sc_spmm_chunk(h, cl, rw, vl, fc) for h in h_chunks]


def _row_spec(f):
    return pl.BlockSpec((ROWS, f), lambda i: (i, 0))


def _full_spec(shape):
    return pl.BlockSpec(shape, lambda i: tuple(0 for _ in shape))


def _pair_spec(f):
    return pl.BlockSpec((2, ROWS, f), lambda i: (0, i, 0))


def _ae_kernel(x, p):
    ws = []
    for name in ("enc1", "enc2", "enc3", "z_layer",
                 "dec1", "dec2", "dec3", "x_bar_layer"):
        ws.append(p[name]["w"])
        ws.append(p[name]["b"].reshape(1, -1))

    def body(x_ref, *refs):
        (w1, b1, w2, b2, w3, b3, wz, bz,
         wd1, bd1, wd2, bd2, wd3, bd3, wx, bx,
         e1_o, e2_o, e3_o, z_o, xb_o) = refs
        xb = x_ref[...]
        e1 = jnp.maximum(_dot(xb, w1[...]) + b1[...], 0.0)
        e2 = jnp.maximum(_dot(e1, w2[...]) + b2[...], 0.0)
        e3 = jnp.maximum(_dot(e2, w3[...]) + b3[...], 0.0)
        z = _dot(e3, wz[...]) + bz[...]
        d1 = jnp.maximum(_dot(z, wd1[...]) + bd1[...], 0.0)
        d2 = jnp.maximum(_dot(d1, wd2[...]) + bd2[...], 0.0)
        d3 = jnp.maximum(_dot(d2, wd3[...]) + bd3[...], 0.0)
        xb_o[...] = _dot(d3, wx[...]) + bx[...]
        e1_o[...] = e1
        e2_o[...] = e2
        e3_o[...] = e3
        z_o[...] = z

    outs = [jax.ShapeDtypeStruct((N, 128), F32),
            jax.ShapeDtypeStruct((N, 256), F32),
            jax.ShapeDtypeStruct((N, 512), F32),
            jax.ShapeDtypeStruct((N, 20), F32),
            jax.ShapeDtypeStruct((N, 128), F32)]
    return pl.pallas_call(
        body,
        grid=(GRID,),
        in_specs=[_row_spec(128)] + [_full_spec(w.shape) for w in ws],
        out_specs=[_row_spec(128), _row_spec(256), _row_spec(512),
                   _row_spec(20), _row_spec(128)],
        out_shape=outs,
    )(x, *ws)


def _mm_chunks(h, w, b=None):
    fi, fo = w.shape
    nc = fo // 128
    ins = [h, w] + ([] if b is None else [b.reshape(1, -1)])

    def body(*refs):
        if b is None:
            h_ref, w_ref = refs[:2]
            outs = refs[2:]
            y = _dot(h_ref[...], w_ref[...])
        else:
            h_ref, w_ref, b_ref = refs[:3]
            outs = refs[3:]
            y = _dot(h_ref[...], w_ref[...]) + b_ref[...]
        for c in range(nc):
            outs[c][...] = y[:, c * 128:(c + 1) * 128]

    in_specs = [_row_spec(fi), _full_spec((fi, fo))]
    if b is not None:
        in_specs.append(_full_spec((1, fo)))
    return pl.pallas_call(
        body,
        grid=(GRID,),
        in_specs=in_specs,
        out_specs=[_row_spec(128)] * nc,
        out_shape=[jax.ShapeDtypeStruct((N, 128), F32)] * nc,
    )(*ins)


def _fusion_a(parts, e, fp, f, relu_g, gpad):
    nc = len(parts)
    ws = [fp["fc1"]["w"], fp["fc1"]["b"].reshape(1, -1),
          fp["fc2"]["w"], fp["fc2"]["b"].reshape(1, -1),
          fp["fc3"]["w"], fp["fc3"]["b"].reshape(1, -1)]
    fcw = fp["fc1"]["w"]

    def body(*refs):
        prefs = refs[:nc]
        e_ref = refs[nc]
        w1, b1, w2, b2, w3, b3, g_o, att_o = refs[nc + 1:]
        gs = []
        for c in range(nc):
            pc = prefs[c][...]
            g = pc[0] + pc[1]
            if relu_g:
                g = jnp.maximum(g, 0.0)
            gs.append(g)
        g = jnp.concatenate(gs, axis=1) if nc > 1 else gs[0]
        eb = e_ref[...]
        gtrim = g[:, :f]
        w1a = w1[...]
        h = _dot(gtrim, w1a[:f]) + _dot(eb, w1a[f:]) + b1[...]
        h = _leaky(h)
        h = _leaky(_dot(h, w2[...]) + b2[...])
        h = jax.nn.sigmoid(_dot(h, w3[...]) + b3[...])
        att = jax.nn.softmax(h, axis=1)
        s = jnp.sum(att, axis=0, keepdims=True)
        att_o[...] = jnp.concatenate(
            [s, jnp.zeros((1, 126), F32)], axis=1).reshape(1, 1, 128)
        if gpad > f:
            g = jnp.concatenate(
                [gtrim, jnp.zeros((ROWS, gpad - f), F32)], axis=1)
        g_o[...] = g

    in_specs = ([_pair_spec(gpad if nc == 1 else 128)] * nc
                + [_row_spec(e.shape[1])]
                + [_full_spec(w.shape) for w in ws])
    return pl.pallas_call(
        body,
        grid=(GRID,),
        in_specs=in_specs,
        out_specs=[_row_spec(gpad),
                   pl.BlockSpec((1, 1, 128), lambda i: (i, 0, 0))],
        out_shape=[jax.ShapeDtypeStruct((N, gpad), F32),
                   jax.ShapeDtypeStruct((GRID, 1, 128), F32)],
    )(*(list(parts) + [e] + ws))


def _fusion_b_mm(attp, g, e, w, f, gpad):
    fi, fo = w.shape
    nc = fo // 128

    def body(att_ref, g_ref, e_ref, w_ref, *outs):
        s = jnp.sum(att_ref[...], axis=0) / float(N)
        w0 = s[:, 0:1]
        w1 = s[:, 1:2]
        h = w0 * g_ref[...][:, :f] + w1 * e_ref[...]
        y = _dot(h, w_ref[...])
        for c in range(nc):
            outs[c][...] = y[:, c * 128:(c + 1) * 128]

    return pl.pallas_call(
        body,
        grid=(GRID,),
        in_specs=[_full_spec((GRID, 1, 128)), _row_spec(gpad),
                  _row_spec(e.shape[1]), _full_spec((fi, fo))],
        out_specs=[_row_spec(128)] * nc,
        out_shape=[jax.ShapeDtypeStruct((N, 128), F32)] * nc,
    )(attp, g, e, w)


def _fusion_b_zi(attp, g, zae):
    def body(att_ref, g_ref, z_ref, out_ref):
        s = jnp.sum(att_ref[...], axis=0) / float(N)
        w0 = s[:, 0:1]
        w1 = s[:, 1:2]
        zi = w0 * g_ref[...][:, :20] + w1 * z_ref[...]
        out_ref[...] = jnp.concatenate(
            [zi, jnp.zeros((ROWS, 108), F32)], axis=1)

    return pl.pallas_call(
        body,
        grid=(GRID,),
        in_specs=[_full_spec((GRID, 1, 128)), _row_spec(128), _row_spec(20)],
        out_specs=_row_spec(128),
        out_shape=jax.ShapeDtypeStruct((N, 128), F32),
    )(attp, g, zae)


def _combine_mm(parts, w, relu_g, fc_in=128):
    nc_in = len(parts)
    fi, fo = w.shape
    nc_out = fo // 128

    def body(*refs):
        prefs = refs[:nc_in]
        w_ref = refs[nc_in]
        outs = refs[nc_in + 1:]
        gs = []
        for c in range(nc_in):
            pc = prefs[c][...]
            g = pc[0] + pc[1]
            if relu_g:
                g = jnp.maximum(g, 0.0)
            gs.append(g)
        h = jnp.concatenate(gs, axis=1) if nc_in > 1 else gs[0]
        y = _dot(h[:, :fi], w_ref[...])
        for c in range(nc_out):
            outs[c][...] = y[:, c * 128:(c + 1) * 128]

    return pl.pallas_call(
        body,
        grid=(GRID,),
        in_specs=[_pair_spec(fc_in)] * nc_in + [_full_spec((fi, fo))],
        out_specs=[_row_spec(128)] * nc_out,
        out_shape=[jax.ShapeDtypeStruct((N, 128), F32)] * nc_out,
    )(*(list(parts) + [w]))


def _combine_relu(parts):
    def body(p_ref, out_ref):
        pc = p_ref[...]
        out_ref[...] = jnp.maximum(pc[0] + pc[1], 0.0)

    return pl.pallas_call(
        body,
        grid=(GRID,),
        in_specs=[_pair_spec(128)],
        out_specs=_row_spec(128),
        out_shape=jax.ShapeDtypeStruct((N, 128), F32),
    )(parts[0])


def _adj_hat(z_hat):
    def body(a_ref, b_ref, out_ref):
        out_ref[...] = jax.nn.sigmoid(
            jnp.dot(a_ref[...], b_ref[...].T, preferred_element_type=F32))

    return pl.pallas_call(
        body,
        grid=(GRID, GRID),
        in_specs=[pl.BlockSpec((ROWS, 128), lambda i, j: (i, 0)),
                  pl.BlockSpec((1024, 128), lambda i, j: (j, 0))],
        out_specs=pl.BlockSpec((ROWS, 1024), lambda i, j: (i, j)),
        out_shape=jax.ShapeDtypeStruct((N, N), F32),
    )(z_hat, z_hat)


def _q_kernel(zl_parts, z_ae, cluster):
    cl = jnp.concatenate([cluster, jnp.zeros((10, 108), F32)], axis=1)

    def tdist(z, c_ref):
        c = c_ref[...]
        zn = jnp.sum(z * z, axis=1, keepdims=True)
        cn = jnp.sum(c * c, axis=1)[None, :]
        d2 = zn + cn - 2.0 * jnp.dot(z, c.T, preferred_element_type=F32)
        q = 1.0 / (1.0 + d2)
        return q / jnp.sum(q, axis=1, keepdims=True)

    def body(p_ref, z_ref, c_ref, q_o, q1_o, zl_o):
        pc = p_ref[...]
        zl = pc[0] + pc[1]
        q_o[...] = tdist(zl, c_ref)
        zae = z_ref[...]
        zae32 = jnp.concatenate([zae, jnp.zeros((ROWS, 108), F32)], axis=1)
        q1_o[...] = tdist(zae32, c_ref)
        zl_o[...] = zl[:, :20]

    return pl.pallas_call(
        body,
        grid=(GRID,),
        in_specs=[_pair_spec(128), _row_spec(20), _full_spec((10, 128))],
        out_specs=[_row_spec(10), _row_spec(10), _row_spec(20)],
        out_shape=[jax.ShapeDtypeStruct((N, 10), F32),
                   jax.ShapeDtypeStruct((N, 10), F32),
                   jax.ShapeDtypeStruct((N, 20), F32)],
    )(zl_parts, z_ae, cl)


@jax.jit
def kernel(x, adj_values, params, edge_index):
    p = params
    col = edge_index[0].reshape(NTILES, EPT)
    row = edge_index[1].reshape(NTILES, EPT)
    val = adj_values.reshape(NTILES, EPT)
    padw = ((0, 0), (0, NBLK * EB - EPT))
    ed3 = (jnp.pad(col, padw).reshape(NTILES, NBLK, EB),
           jnp.pad(row, padw).reshape(NTILES, NBLK, EB),
           jnp.pad(val, padw).reshape(NTILES, NBLK, EB))

    e1, e2, e3, z_ae, x_bar = _ae_kernel(x, p)

    m1 = _mm_chunks(x, p["gnn1"])
    p1 = _sc_spmm(m1, ed3)
    g1, att1 = _fusion_a(p1, e1, p["fuse1"], 128, True, 128)
    m2 = _fusion_b_mm(att1, g1, e1, p["gnn2"], 128, 128)
    p2 = _sc_spmm(m2, ed3)
    g2, att2 = _fusion_a(p2, e2, p["fuse2"], 256, True, 256)
    m3 = _fusion_b_mm(att2, g2, e2, p["gnn3"], 256, 256)
    p3 = _sc_spmm(m3, ed3)
    g3, att3 = _fusion_a(p3, e3, p["fuse3"], 512, True, 512)
    gnn4_pad = jnp.concatenate(
        [p["gnn4"], jnp.zeros((512, 108), F32)], axis=1)
    m4 = _fusion_b_zi32(att3, g3, e3, gnn4_pad)
    p4 = _sc_spmm([m4], ed3)
    zg, att4 = _fusion_a(p4, z_ae, p["fuse4"], 20, False, 128)
    z_i = _fusion_b_zi(att4, zg, z_ae)
    pl_ = _sc_spmm([z_i], ed3)

    gnn5_pad = jnp.concatenate(
        [p["gnn5"], jnp.zeros((108, 512), F32)], axis=0)
    m5 = _combine_mm(p4, gnn5_pad, False)
    p5 = _sc_spmm(m5, ed3)
    m6 = _combine_mm(p5, p["gnn6"], True)
    p6 = _sc_spmm(m6, ed3)
    m7 = _combine_mm(p6, p["gnn7"], True)
    p7 = _sc_spmm(m7, ed3)
    m8 = _combine_mm(p7, p["gnn8"], True)
    p8 = _sc_spmm(m8, ed3)
    z_hat = _combine_relu(p8)

    adj_hat = _adj_hat(z_hat)
    q, q1, z_l = _q_kernel(pl_[0], z_ae, p["cluster"])
    return (x_bar, z_hat, adj_hat, z_ae, q, q1, z_l)


def _fusion_b_zi32(attp, g, e, w32):
    def body(att_ref, g_ref, e_ref, w_ref, out_ref):
        s = jnp.sum(att_ref[...], axis=0) / float(N)
        w0 = s[:, 0:1]
        w1 = s[:, 1:2]
        h = w0 * g_ref[...] + w1 * e_ref[...]
        out_ref[...] = _dot(h, w_ref[...])

    return pl.pallas_call(
        body,
        grid=(GRID,),
        in_specs=[_full_spec((GRID, 1, 128)), _row_spec(512), _row_spec(512),
                  _full_spec((512, 128))],
        out_specs=_row_spec(128),
        out_shape=jax.ShapeDtypeStruct((N, 128), F32),
    )(attp, g, e, w32)


# speedup vs baseline: 1.7624x; 1.7624x over previous
import functools

import jax
import jax.numpy as jnp
from jax import lax
from jax.experimental import pallas as pl
from jax.experimental.pallas import tpu as pltpu
from jax.experimental.pallas import tpu_sc as plsc

N = 10000
E = 160000
NP = 10112
ROWS = 1000
GRID = N // ROWS
NTILES = 32
EPT = E // NTILES
EB = 128
NBLK = 40
F32 = jnp.float32


def _leaky(v):
    return jnp.where(v >= 0, v, 0.01 * v)


def _dot(a, b):
    return jnp.dot(a, b, preferred_element_type=F32)


def _sc_spmm_chunk(h, cl3, rw3, vl3, fc):
    mesh = plsc.VectorSubcoreMesh(core_axis_name="c", subcore_axis_name="s")
    rpt = NP // 16
    nlane = fc // 16

    def body(h_hbm, cl_hbm, rw_hbm, vl_hbm, out_hbm,
             col_v, row_v, val_v, gat_v, acc_sh, sem):
        cid = lax.axis_index("c")
        sid = lax.axis_index("s")
        wid = cid * 16 + sid

        def zrow(r, _):
            for f in range(nlane):
                gat_v[r, pl.ds(16 * f, 16)] = jnp.zeros((16,), F32)
            return 0
        lax.fori_loop(0, EB, zrow, 0)
        base = sid * rpt
        nfull = rpt // EB
        rem = rpt - nfull * EB
        for k in range(nfull):
            pltpu.sync_copy(gat_v, acc_sh.at[pl.ds(base + k * EB, EB)])
        pltpu.sync_copy(gat_v.at[pl.ds(0, rem)],
                        acc_sh.at[pl.ds(base + nfull * EB, rem)])
        plsc.subcore_barrier()

        def blk(b, _):
            pltpu.sync_copy(cl_hbm.at[wid, b], col_v)
            pltpu.sync_copy(rw_hbm.at[wid, b], row_v)
            pltpu.sync_copy(vl_hbm.at[wid, b], val_v)
            pltpu.async_copy(h_hbm.at[col_v], gat_v, sem).wait()

            def scale(jj, _):
                val16 = val_v[pl.ds(16 * jj, 16)]
                for l in range(16):
                    v = val16[l]
                    for f in range(nlane):
                        sl = pl.ds(16 * f, 16)
                        gat_v[16 * jj + l, sl] = gat_v[16 * jj + l, sl] * v
                return 0
            lax.fori_loop(0, EB // 16, scale, 0)
            pltpu.sync_copy(gat_v, acc_sh.at[row_v], add=True)
            return 0
        lax.fori_loop(0, NBLK, blk, 0)
        plsc.subcore_barrier()

        for k in range(nfull):
            pltpu.sync_copy(acc_sh.at[pl.ds(base + k * EB, EB)],
                            out_hbm.at[cid, pl.ds(base + k * EB, EB)])
        pltpu.sync_copy(acc_sh.at[pl.ds(base + nfull * EB, rem)],
                        out_hbm.at[cid, pl.ds(base + nfull * EB, rem)])

    run = pl.kernel(
        body,
        out_type=jax.ShapeDtypeStruct((2, NP, fc), F32),
        mesh=mesh,
        scratch_types=[
            pltpu.VMEM((EB,), jnp.int32),
            pltpu.VMEM((EB,), jnp.int32),
            pltpu.VMEM((EB,), F32),
            pltpu.VMEM((EB, fc), F32),
            pltpu.VMEM_SHARED((NP, fc), F32),
            pltpu.SemaphoreType.DMA,
        ],
    )
    return run(h, cl3, rw3, vl3)


def _sc_spmm(h_chunks, ed3, fc=128):
    cl, rw, vl = ed3
    return [_sc_spmm_chunk(h, cl, rw, vl, fc) for h in h_chunks]


def _row_spec(f):
    return pl.BlockSpec((ROWS, f), lambda i: (i, 0))


def _full_spec(shape):
    return pl.BlockSpec(shape, lambda i: tuple(0 for _ in shape))


def _pair_spec(f):
    return pl.BlockSpec((2, ROWS, f), lambda i: (0, i, 0))


def _ae_kernel(x, p):
    ws = []
    for name in ("enc1", "enc2", "enc3", "z_layer",
                 "dec1", "dec2", "dec3", "x_bar_layer"):
        ws.append(p[name]["w"])
        ws.append(p[name]["b"].reshape(1, -1))

    def body(x_ref, *refs):
        (w1, b1, w2, b2, w3, b3, wz, bz,
         wd1, bd1, wd2, bd2, wd3, bd3, wx, bx,
         e1_o, e2_o, e3_o, z_o, xb_o) = refs
        xb = x_ref[...]
        e1 = jnp.maximum(_dot(xb, w1[...]) + b1[...], 0.0)
        e2 = jnp.maximum(_dot(e1, w2[...]) + b2[...], 0.0)
        e3 = jnp.maximum(_dot(e2, w3[...]) + b3[...], 0.0)
        z = _dot(e3, wz[...]) + bz[...]
        d1 = jnp.maximum(_dot(z, wd1[...]) + bd1[...], 0.0)
        d2 = jnp.maximum(_dot(d1, wd2[...]) + bd2[...], 0.0)
        d3 = jnp.maximum(_dot(d2, wd3[...]) + bd3[...], 0.0)
        xb_o[...] = _dot(d3, wx[...]) + bx[...]
        e1_o[...] = e1
        e2_o[...] = e2
        e3_o[...] = e3
        z_o[...] = z

    outs = [jax.ShapeDtypeStruct((N, 128), F32),
            jax.ShapeDtypeStruct((N, 256), F32),
            jax.ShapeDtypeStruct((N, 512), F32),
            jax.ShapeDtypeStruct((N, 20), F32),
            jax.ShapeDtypeStruct((N, 128), F32)]
    return pl.pallas_call(
        body,
        grid=(GRID,),
        in_specs=[_row_spec(128)] + [_full_spec(w.shape) for w in ws],
        out_specs=[_row_spec(128), _row_spec(256), _row_spec(512),
                   _row_spec(20), _row_spec(128)],
        out_shape=outs,
    )(x, *ws)


def _mm_chunks(h, w, b=None):
    fi, fo = w.shape
    nc = fo // 128
    ins = [h, w] + ([] if b is None else [b.reshape(1, -1)])

    def body(*refs):
        if b is None:
            h_ref, w_ref = refs[:2]
            outs = refs[2:]
            y = _dot(h_ref[...], w_ref[...])
        else:
            h_ref, w_ref, b_ref = refs[:3]
            outs = refs[3:]
            y = _dot(h_ref[...], w_ref[...]) + b_ref[...]
        for c in range(nc):
            outs[c][...] = y[:, c * 128:(c + 1) * 128]

    in_specs = [_row_spec(fi), _full_spec((fi, fo))]
    if b is not None:
        in_specs.append(_full_spec((1, fo)))
    return pl.pallas_call(
        body,
        grid=(GRID,),
        in_specs=in_specs,
        out_specs=[_row_spec(128)] * nc,
        out_shape=[jax.ShapeDtypeStruct((N, 128), F32)] * nc,
    )(*ins)


def _fusion_a(parts, e, fp, f, relu_g, gpad):
    nc = len(parts)
    ws = [fp["fc1"]["w"], fp["fc1"]["b"].reshape(1, -1),
          fp["fc2"]["w"], fp["fc2"]["b"].reshape(1, -1),
          fp["fc3"]["w"], fp["fc3"]["b"].reshape(1, -1)]
    fcw = fp["fc1"]["w"]

    def body(*refs):
        prefs = refs[:nc]
        e_ref = refs[nc]
        w1, b1, w2, b2, w3, b3, g_o, att_o = refs[nc + 1:]
        gs = []
        for c in range(nc):
            pc = prefs[c][...]
            g = pc[0] + pc[1]
            if relu_g:
                g = jnp.maximum(g, 0.0)
            gs.append(g)
        g = jnp.concatenate(gs, axis=1) if nc > 1 else gs[0]
        eb = e_ref[...]
        gtrim = g[:, :f]
        w1a = w1[...]
        h = _dot(gtrim, w1a[:f]) + _dot(eb, w1a[f:]) + b1[...]
        h = _leaky(h)
        h = _leaky(_dot(h, w2[...]) + b2[...])
        h = jax.nn.sigmoid(_dot(h, w3[...]) + b3[...])
        att = jax.nn.softmax(h, axis=1)
        s = jnp.sum(att, axis=0, keepdims=True)
        att_o[...] = jnp.concatenate(
            [s, jnp.zeros((1, 126), F32)], axis=1).reshape(1, 1, 128)
        if gpad > f:
            g = jnp.concatenate(
                [gtrim, jnp.zeros((ROWS, gpad - f), F32)], axis=1)
        g_o[...] = g

    in_specs = ([_pair_spec(gpad if nc == 1 else 128)] * nc
                + [_row_spec(e.shape[1])]
                + [_full_spec(w.shape) for w in ws])
    return pl.pallas_call(
        body,
        grid=(GRID,),
        in_specs=in_specs,
        out_specs=[_row_spec(gpad),
                   pl.BlockSpec((1, 1, 128), lambda i: (i, 0, 0))],
        out_shape=[jax.ShapeDtypeStruct((N, gpad), F32),
                   jax.ShapeDtypeStruct((GRID, 1, 128), F32)],
    )(*(list(parts) + [e] + ws))


def _fusion_b_mm(attp, g, e, w, f, gpad):
    fi, fo = w.shape
    nc = fo // 128

    def body(att_ref, g_ref, e_ref, w_ref, *outs):
        s = jnp.sum(att_ref[...], axis=0) / float(N)
        w0 = s[:, 0:1]
        w1 = s[:, 1:2]
        h = w0 * g_ref[...][:, :f] + w1 * e_ref[...]
        y = _dot(h, w_ref[...])
        for c in range(nc):
            outs[c][...] = y[:, c * 128:(c + 1) * 128]

    return pl.pallas_call(
        body,
        grid=(GRID,),
        in_specs=[_full_spec((GRID, 1, 128)), _row_spec(gpad),
                  _row_spec(e.shape[1]), _full_spec((fi, fo))],
        out_specs=[_row_spec(128)] * nc,
        out_shape=[jax.ShapeDtypeStruct((N, 128), F32)] * nc,
    )(attp, g, e, w)


def _fusion_b_zi(attp, g, zae):
    def body(att_ref, g_ref, z_ref, out_ref):
        s = jnp.sum(att_ref[...], axis=0) / float(N)
        w0 = s[:, 0:1]
        w1 = s[:, 1:2]
        zi = w0 * g_ref[...][:, :20] + w1 * z_ref[...]
        out_ref[...] = jnp.concatenate(
            [zi, jnp.zeros((ROWS, 108), F32)], axis=1)

    return pl.pallas_call(
        body,
        grid=(GRID,),
        in_specs=[_full_spec((GRID, 1, 128)), _row_spec(128), _row_spec(20)],
        out_specs=_row_spec(128),
        out_shape=jax.ShapeDtypeStruct((N, 128), F32),
    )(attp, g, zae)


def _combine_mm(parts, w, relu_g, fc_in=128):
    nc_in = len(parts)
    fi, fo = w.shape
    nc_out = fo // 128

    def body(*refs):
        prefs = refs[:nc_in]
        w_ref = refs[nc_in]
        outs = refs[nc_in + 1:]
        gs = []
        for c in range(nc_in):
            pc = prefs[c][...]
            g = pc[0] + pc[1]
            if relu_g:
                g = jnp.maximum(g, 0.0)
            gs.append(g)
        h = jnp.concatenate(gs, axis=1) if nc_in > 1 else gs[0]
        y = _dot(h[:, :fi], w_ref[...])
        for c in range(nc_out):
            outs[c][...] = y[:, c * 128:(c + 1) * 128]

    return pl.pallas_call(
        body,
        grid=(GRID,),
        in_specs=[_pair_spec(fc_in)] * nc_in + [_full_spec((fi, fo))],
        out_specs=[_row_spec(128)] * nc_out,
        out_shape=[jax.ShapeDtypeStruct((N, 128), F32)] * nc_out,
    )(*(list(parts) + [w]))


def _combine_relu(parts):
    def body(p_ref, out_ref):
        pc = p_ref[...]
        out_ref[...] = jnp.maximum(pc[0] + pc[1], 0.0)

    return pl.pallas_call(
        body,
        grid=(GRID,),
        in_specs=[_pair_spec(128)],
        out_specs=_row_spec(128),
        out_shape=jax.ShapeDtypeStruct((N, 128), F32),
    )(parts[0])


def _adj_hat(z_hat):
    def body(a_ref, b_ref, out_ref):
        out_ref[...] = jax.nn.sigmoid(
            jnp.dot(a_ref[...], b_ref[...].T, preferred_element_type=F32))

    return pl.pallas_call(
        body,
        grid=(GRID, GRID),
        in_specs=[pl.BlockSpec((ROWS, 128), lambda i, j: (i, 0)),
                  pl.BlockSpec((1024, 128), lambda i, j: (j, 0))],
        out_specs=pl.BlockSpec((ROWS, 1024), lambda i, j: (i, j)),
        out_shape=jax.ShapeDtypeStruct((N, N), F32),
    )(z_hat, z_hat)


def _q_kernel(zl_parts, z_ae, cluster):
    cl = jnp.concatenate([cluster, jnp.zeros((10, 108), F32)], axis=1)

    def tdist(z, c_ref):
        c = c_ref[...]
        zn = jnp.sum(z * z, axis=1, keepdims=True)
        cn = jnp.sum(c * c, axis=1)[None, :]
        d2 = zn + cn - 2.0 * jnp.dot(z, c.T, preferred_element_type=F32)
        q = 1.0 / (1.0 + d2)
        return q / jnp.sum(q, axis=1, keepdims=True)

    def body(p_ref, z_ref, c_ref, q_o, q1_o, zl_o):
        pc = p_ref[...]
        zl = pc[0] + pc[1]
        q_o[...] = tdist(zl, c_ref)
        zae = z_ref[...]
        zae32 = jnp.concatenate([zae, jnp.zeros((ROWS, 108), F32)], axis=1)
        q1_o[...] = tdist(zae32, c_ref)
        zl_o[...] = zl[:, :20]

    return pl.pallas_call(
        body,
        grid=(GRID,),
        in_specs=[_pair_spec(128), _row_spec(20), _full_spec((10, 128))],
        out_specs=[_row_spec(10), _row_spec(10), _row_spec(20)],
        out_shape=[jax.ShapeDtypeStruct((N, 10), F32),
                   jax.ShapeDtypeStruct((N, 10), F32),
                   jax.ShapeDtypeStruct((N, 20), F32)],
    )(zl_parts, z_ae, cl)


@jax.jit
def kernel(x, adj_values, params, edge_index):
    p = params
    col = edge_index[0].reshape(NTILES, EPT)
    row = edge_index[1].reshape(NTILES, EPT)
    val = adj_values.reshape(NTILES, EPT)
    padw = ((0, 0), (0, NBLK * EB - EPT))
    ed3 = (jnp.pad(col, padw).reshape(NTILES, NBLK, EB),
           jnp.pad(row, padw).reshape(NTILES, NBLK, EB),
           jnp.pad(val, padw).reshape(NTILES, NBLK, EB))

    e1, e2, e3, z_ae, x_bar = _ae_kernel(x, p)

    m1 = _mm_chunks(x, p["gnn1"])
    p1 = _sc_spmm(m1, ed3)
    g1, att1 = _fusion_a(p1, e1, p["fuse1"], 128, True, 128)
    m2 = _fusion_b_mm(att1, g1, e1, p["gnn2"], 128, 128)
    p2 = _sc_spmm(m2, ed3)
    g2, att2 = _fusion_a(p2, e2, p["fuse2"], 256, True, 256)
    m3 = _fusion_b_mm(att2, g2, e2, p["gnn3"], 256, 256)
    p3 = _sc_spmm(m3, ed3)
    g3, att3 = _fusion_a(p3, e3, p["fuse3"], 512, True, 512)
    gnn4_pad = jnp.concatenate(
        [p["gnn4"], jnp.zeros((512, 108), F32)], axis=1)
    m4 = _fusion_b_zi32(att3, g3, e3, gnn4_pad)
    p4 = _sc_spmm([m4], ed3)
    zg, att4 = _fusion_a(p4, z_ae, p["fuse4"], 20, False, 128)
    z_i = _fusion_b_zi(att4, zg, z_ae)
    pl_ = _sc_spmm([z_i], ed3)

    gnn5_pad = jnp.concatenate(
        [p["gnn5"], jnp.zeros((108, 512), F32)], axis=0)
    m5 = _combine_mm(p4, gnn5_pad, False)
    p5 = _sc_spmm(m5, ed3)
    m6 = _combine_mm(p5, p["gnn6"], True)
    p6 = _sc_spmm(m6, ed3)
    m7 = _combine_mm(p6, p["gnn7"], True)
    p7 = _sc_spmm(m7, ed3)
    m8 = _combine_mm(p7, p["gnn8"], True)
    p8 = _sc_spmm(m8, ed3)
    z_hat = _combine_relu(p8)

    adj_hat = _adj_hat(z_hat)
    q, q1, z_l = _q_kernel(pl_[0], z_ae, p["cluster"])
    return (x_bar, z_hat, adj_hat, z_ae, q, q1, z_l)


def _fusion_b_zi32(attp, g, e, w32):
    def body(att_ref, g_ref, e_ref, w_ref, out_ref):
        s = jnp.sum(att_ref[...], axis=0) / float(N)
        w0 = s[:, 0:1]
        w1 = s[:, 1:2]
        h = w0 * g_ref[...] + w1 * e_ref[...]
        out_ref[...] = _dot(h, w_ref[...])

    return pl.pallas_call(
        body,
        grid=(GRID,),
        in_specs=[_full_spec((GRID, 1, 128)), _row_spec(512), _row_spec(512),
                  _full_spec((512, 128))],
        out_specs=_row_spec(128),
        out_shape=jax.ShapeDtypeStruct((N, 128), F32),
    )(attp, g, e, w32)


# speedup vs baseline: 1.9518x; 1.1075x over previous
import functools

import jax
import jax.numpy as jnp
from jax import lax
from jax.experimental import pallas as pl
from jax.experimental.pallas import tpu as pltpu
from jax.experimental.pallas import tpu_sc as plsc

N = 10000
E = 160000
NP = 10112
ROWS = 1000
GRID = N // ROWS
NTILES = 32
EPT = E // NTILES
EB = 128
NBLK = 40
F32 = jnp.float32


def _leaky(v):
    return jnp.where(v >= 0, v, 0.01 * v)


def _dot(a, b):
    return jnp.dot(a, b, preferred_element_type=F32)


def _sc_spmm_chunk(h, cl3, rw3, vl3, fc):
    mesh = plsc.VectorSubcoreMesh(core_axis_name="c", subcore_axis_name="s")
    rpt = NP // 16
    nlane = fc // 16

    def body(h_hbm, cl_hbm, rw_hbm, vl_hbm, out_hbm,
             col_v, row_v, val_v, gat_v, acc_sh, sem):
        cid = lax.axis_index("c")
        sid = lax.axis_index("s")
        wid = cid * 16 + sid

        def zrow(r, _):
            for f in range(nlane):
                gat_v[r, pl.ds(16 * f, 16)] = jnp.zeros((16,), F32)
            return 0
        lax.fori_loop(0, EB, zrow, 0)
        base = sid * rpt
        nfull = rpt // EB
        rem = rpt - nfull * EB
        for k in range(nfull):
            pltpu.sync_copy(gat_v, acc_sh.at[pl.ds(base + k * EB, EB)])
        pltpu.sync_copy(gat_v.at[pl.ds(0, rem)],
                        acc_sh.at[pl.ds(base + nfull * EB, rem)])
        plsc.subcore_barrier()

        def blk(b, _):
            h1 = pltpu.async_copy(cl_hbm.at[wid, b], col_v, sem)
            h2 = pltpu.async_copy(rw_hbm.at[wid, b], row_v, sem)
            h3 = pltpu.async_copy(vl_hbm.at[wid, b], val_v, sem)
            h1.wait()
            h2.wait()
            h3.wait()
            pltpu.async_copy(h_hbm.at[col_v], gat_v, sem).wait()

            def scale(jj, _):
                val16 = val_v[pl.ds(16 * jj, 16)]
                for l in range(16):
                    v = val16[l]
                    for f in range(nlane):
                        sl = pl.ds(16 * f, 16)
                        gat_v[16 * jj + l, sl] = gat_v[16 * jj + l, sl] * v
                return 0
            lax.fori_loop(0, EB // 16, scale, 0)
            pltpu.sync_copy(gat_v, acc_sh.at[row_v], add=True)
            return 0
        lax.fori_loop(0, NBLK, blk, 0)
        plsc.subcore_barrier()

        for k in range(nfull):
            pltpu.sync_copy(acc_sh.at[pl.ds(base + k * EB, EB)],
                            out_hbm.at[cid, pl.ds(base + k * EB, EB)])
        pltpu.sync_copy(acc_sh.at[pl.ds(base + nfull * EB, rem)],
                        out_hbm.at[cid, pl.ds(base + nfull * EB, rem)])

    run = pl.kernel(
        body,
        out_type=jax.ShapeDtypeStruct((2, NP, fc), F32),
        mesh=mesh,
        scratch_types=[
            pltpu.VMEM((EB,), jnp.int32),
            pltpu.VMEM((EB,), jnp.int32),
            pltpu.VMEM((EB,), F32),
            pltpu.VMEM((EB, fc), F32),
            pltpu.VMEM_SHARED((NP, fc), F32),
            pltpu.SemaphoreType.DMA,
        ],
    )
    return run(h, cl3, rw3, vl3)


def _sc_spmm(h_chunks, ed3, fc=128):
    cl, rw, vl = ed3
    return [_sc_spmm_chunk(h, cl, rw, vl, fc) for h in h_chunks]


def _row_spec(f):
    return pl.BlockSpec((ROWS, f), lambda i: (i, 0))


def _full_spec(shape):
    return pl.BlockSpec(shape, lambda i: tuple(0 for _ in shape))


def _pair_spec(f):
    return pl.BlockSpec((2, ROWS, f), lambda i: (0, i, 0))


def _ae_kernel(x, p):
    ws = []
    for name in ("enc1", "enc2", "enc3", "z_layer",
                 "dec1", "dec2", "dec3", "x_bar_layer"):
        ws.append(p[name]["w"])
        ws.append(p[name]["b"].reshape(1, -1))

    def body(x_ref, *refs):
        (w1, b1, w2, b2, w3, b3, wz, bz,
         wd1, bd1, wd2, bd2, wd3, bd3, wx, bx,
         e1_o, e2_o, e3_o, z_o, xb_o) = refs
        xb = x_ref[...]
        e1 = jnp.maximum(_dot(xb, w1[...]) + b1[...], 0.0)
        e2 = jnp.maximum(_dot(e1, w2[...]) + b2[...], 0.0)
        e3 = jnp.maximum(_dot(e2, w3[...]) + b3[...], 0.0)
        z = _dot(e3, wz[...]) + bz[...]
        d1 = jnp.maximum(_dot(z, wd1[...]) + bd1[...], 0.0)
        d2 = jnp.maximum(_dot(d1, wd2[...]) + bd2[...], 0.0)
        d3 = jnp.maximum(_dot(d2, wd3[...]) + bd3[...], 0.0)
        xb_o[...] = _dot(d3, wx[...]) + bx[...]
        e1_o[...] = e1
        e2_o[...] = e2
        e3_o[...] = e3
        z_o[...] = z

    outs = [jax.ShapeDtypeStruct((N, 128), F32),
            jax.ShapeDtypeStruct((N, 256), F32),
            jax.ShapeDtypeStruct((N, 512), F32),
            jax.ShapeDtypeStruct((N, 20), F32),
            jax.ShapeDtypeStruct((N, 128), F32)]
    return pl.pallas_call(
        body,
        grid=(GRID,),
        in_specs=[_row_spec(128)] + [_full_spec(w.shape) for w in ws],
        out_specs=[_row_spec(128), _row_spec(256), _row_spec(512),
                   _row_spec(20), _row_spec(128)],
        out_shape=outs,
    )(x, *ws)


def _mm_chunks(h, w, b=None):
    fi, fo = w.shape
    nc = fo // 128
    ins = [h, w] + ([] if b is None else [b.reshape(1, -1)])

    def body(*refs):
        if b is None:
            h_ref, w_ref = refs[:2]
            outs = refs[2:]
            y = _dot(h_ref[...], w_ref[...])
        else:
            h_ref, w_ref, b_ref = refs[:3]
            outs = refs[3:]
            y = _dot(h_ref[...], w_ref[...]) + b_ref[...]
        for c in range(nc):
            outs[c][...] = y[:, c * 128:(c + 1) * 128]

    in_specs = [_row_spec(fi), _full_spec((fi, fo))]
    if b is not None:
        in_specs.append(_full_spec((1, fo)))
    return pl.pallas_call(
        body,
        grid=(GRID,),
        in_specs=in_specs,
        out_specs=[_row_spec(128)] * nc,
        out_shape=[jax.ShapeDtypeStruct((N, 128), F32)] * nc,
    )(*ins)


def _fusion_a(parts, e, fp, f, relu_g, gpad):
    nc = len(parts)
    ws = [fp["fc1"]["w"], fp["fc1"]["b"].reshape(1, -1),
          fp["fc2"]["w"], fp["fc2"]["b"].reshape(1, -1),
          fp["fc3"]["w"], fp["fc3"]["b"].reshape(1, -1)]
    fcw = fp["fc1"]["w"]

    def body(*refs):
        prefs = refs[:nc]
        e_ref = refs[nc]
        w1, b1, w2, b2, w3, b3, g_o, att_o = refs[nc + 1:]
        gs = []
        for c in range(nc):
            pc = prefs[c][...]
            g = pc[0] + pc[1]
            if relu_g:
                g = jnp.maximum(g, 0.0)
            gs.append(g)
        g = jnp.concatenate(gs, axis=1) if nc > 1 else gs[0]
        eb = e_ref[...]
        gtrim = g[:, :f]
        w1a = w1[...]
        h = _dot(gtrim, w1a[:f]) + _dot(eb, w1a[f:]) + b1[...]
        h = _leaky(h)
        h = _leaky(_dot(h, w2[...]) + b2[...])
        h = jax.nn.sigmoid(_dot(h, w3[...]) + b3[...])
        att = jax.nn.softmax(h, axis=1)
        s = jnp.sum(att, axis=0, keepdims=True)
        att_o[...] = jnp.concatenate(
            [s, jnp.zeros((1, 126), F32)], axis=1).reshape(1, 1, 128)
        if gpad > f:
            g = jnp.concatenate(
                [gtrim, jnp.zeros((ROWS, gpad - f), F32)], axis=1)
        g_o[...] = g

    in_specs = ([_pair_spec(gpad if nc == 1 else 128)] * nc
                + [_row_spec(e.shape[1])]
                + [_full_spec(w.shape) for w in ws])
    return pl.pallas_call(
        body,
        grid=(GRID,),
        in_specs=in_specs,
        out_specs=[_row_spec(gpad),
                   pl.BlockSpec((1, 1, 128), lambda i: (i, 0, 0))],
        out_shape=[jax.ShapeDtypeStruct((N, gpad), F32),
                   jax.ShapeDtypeStruct((GRID, 1, 128), F32)],
    )(*(list(parts) + [e] + ws))


def _fusion_b_mm(attp, g, e, w, f, gpad):
    fi, fo = w.shape
    nc = fo // 128

    def body(att_ref, g_ref, e_ref, w_ref, *outs):
        s = jnp.sum(att_ref[...], axis=0) / float(N)
        w0 = s[:, 0:1]
        w1 = s[:, 1:2]
        h = w0 * g_ref[...][:, :f] + w1 * e_ref[...]
        y = _dot(h, w_ref[...])
        for c in range(nc):
            outs[c][...] = y[:, c * 128:(c + 1) * 128]

    return pl.pallas_call(
        body,
        grid=(GRID,),
        in_specs=[_full_spec((GRID, 1, 128)), _row_spec(gpad),
                  _row_spec(e.shape[1]), _full_spec((fi, fo))],
        out_specs=[_row_spec(128)] * nc,
        out_shape=[jax.ShapeDtypeStruct((N, 128), F32)] * nc,
    )(attp, g, e, w)


def _fusion_b_zi(attp, g, zae):
    def body(att_ref, g_ref, z_ref, out_ref):
        s = jnp.sum(att_ref[...], axis=0) / float(N)
        w0 = s[:, 0:1]
        w1 = s[:, 1:2]
        zi = w0 * g_ref[...][:, :20] + w1 * z_ref[...]
        out_ref[...] = jnp.concatenate(
            [zi, jnp.zeros((ROWS, 108), F32)], axis=1)

    return pl.pallas_call(
        body,
        grid=(GRID,),
        in_specs=[_full_spec((GRID, 1, 128)), _row_spec(128), _row_spec(20)],
        out_specs=_row_spec(128),
        out_shape=jax.ShapeDtypeStruct((N, 128), F32),
    )(attp, g, zae)


def _combine_mm(parts, w, relu_g, fc_in=128):
    nc_in = len(parts)
    fi, fo = w.shape
    nc_out = fo // 128

    def body(*refs):
        prefs = refs[:nc_in]
        w_ref = refs[nc_in]
        outs = refs[nc_in + 1:]
        gs = []
        for c in range(nc_in):
            pc = prefs[c][...]
            g = pc[0] + pc[1]
            if relu_g:
                g = jnp.maximum(g, 0.0)
            gs.append(g)
        h = jnp.concatenate(gs, axis=1) if nc_in > 1 else gs[0]
        y = _dot(h[:, :fi], w_ref[...])
        for c in range(nc_out):
            outs[c][...] = y[:, c * 128:(c + 1) * 128]

    return pl.pallas_call(
        body,
        grid=(GRID,),
        in_specs=[_pair_spec(fc_in)] * nc_in + [_full_spec((fi, fo))],
        out_specs=[_row_spec(128)] * nc_out,
        out_shape=[jax.ShapeDtypeStruct((N, 128), F32)] * nc_out,
    )(*(list(parts) + [w]))


def _combine_relu(parts):
    def body(p_ref, out_ref):
        pc = p_ref[...]
        out_ref[...] = jnp.maximum(pc[0] + pc[1], 0.0)

    return pl.pallas_call(
        body,
        grid=(GRID,),
        in_specs=[_pair_spec(128)],
        out_specs=_row_spec(128),
        out_shape=jax.ShapeDtypeStruct((N, 128), F32),
    )(parts[0])


def _adj_hat(z_hat):
    def body(a_ref, b_ref, out_ref):
        out_ref[...] = jax.nn.sigmoid(
            jnp.dot(a_ref[...], b_ref[...].T, preferred_element_type=F32))

    return pl.pallas_call(
        body,
        grid=(GRID, GRID),
        in_specs=[pl.BlockSpec((ROWS, 128), lambda i, j: (i, 0)),
                  pl.BlockSpec((1024, 128), lambda i, j: (j, 0))],
        out_specs=pl.BlockSpec((ROWS, 1024), lambda i, j: (i, j)),
        out_shape=jax.ShapeDtypeStruct((N, N), F32),
    )(z_hat, z_hat)


def _q_kernel(zl_parts, z_ae, cluster):
    cl = jnp.concatenate([cluster, jnp.zeros((10, 108), F32)], axis=1)

    def tdist(z, c_ref):
        c = c_ref[...]
        zn = jnp.sum(z * z, axis=1, keepdims=True)
        cn = jnp.sum(c * c, axis=1)[None, :]
        d2 = zn + cn - 2.0 * jnp.dot(z, c.T, preferred_element_type=F32)
        q = 1.0 / (1.0 + d2)
        return q / jnp.sum(q, axis=1, keepdims=True)

    def body(p_ref, z_ref, c_ref, q_o, q1_o, zl_o):
        pc = p_ref[...]
        zl = pc[0] + pc[1]
        q_o[...] = tdist(zl, c_ref)
        zae = z_ref[...]
        zae32 = jnp.concatenate([zae, jnp.zeros((ROWS, 108), F32)], axis=1)
        q1_o[...] = tdist(zae32, c_ref)
        zl_o[...] = zl[:, :20]

    return pl.pallas_call(
        body,
        grid=(GRID,),
        in_specs=[_pair_spec(128), _row_spec(20), _full_spec((10, 128))],
        out_specs=[_row_spec(10), _row_spec(10), _row_spec(20)],
        out_shape=[jax.ShapeDtypeStruct((N, 10), F32),
                   jax.ShapeDtypeStruct((N, 10), F32),
                   jax.ShapeDtypeStruct((N, 20), F32)],
    )(zl_parts, z_ae, cl)


@jax.jit
def kernel(x, adj_values, params, edge_index):
    p = params
    col = edge_index[0].reshape(NTILES, EPT)
    row = edge_index[1].reshape(NTILES, EPT)
    val = adj_values.reshape(NTILES, EPT)
    padw = ((0, 0), (0, NBLK * EB - EPT))
    ed3 = (jnp.pad(col, padw).reshape(NTILES, NBLK, EB),
           jnp.pad(row, padw).reshape(NTILES, NBLK, EB),
           jnp.pad(val, padw).reshape(NTILES, NBLK, EB))

    e1, e2, e3, z_ae, x_bar = _ae_kernel(x, p)

    m1 = _mm_chunks(x, p["gnn1"])
    p1 = _sc_spmm(m1, ed3)
    g1, att1 = _fusion_a(p1, e1, p["fuse1"], 128, True, 128)
    m2 = _fusion_b_mm(att1, g1, e1, p["gnn2"], 128, 128)
    p2 = _sc_spmm(m2, ed3)
    g2, att2 = _fusion_a(p2, e2, p["fuse2"], 256, True, 256)
    m3 = _fusion_b_mm(att2, g2, e2, p["gnn3"], 256, 256)
    p3 = _sc_spmm(m3, ed3)
    g3, att3 = _fusion_a(p3, e3, p["fuse3"], 512, True, 512)
    gnn4_pad = jnp.concatenate(
        [p["gnn4"], jnp.zeros((512, 108), F32)], axis=1)
    m4 = _fusion_b_zi32(att3, g3, e3, gnn4_pad)
    p4 = _sc_spmm([m4], ed3)
    zg, att4 = _fusion_a(p4, z_ae, p["fuse4"], 20, False, 128)
    z_i = _fusion_b_zi(att4, zg, z_ae)
    pl_ = _sc_spmm([z_i], ed3)

    gnn5_pad = jnp.concatenate(
        [p["gnn5"], jnp.zeros((108, 512), F32)], axis=0)
    m5 = _combine_mm(p4, gnn5_pad, False)
    p5 = _sc_spmm(m5, ed3)
    m6 = _combine_mm(p5, p["gnn6"], True)
    p6 = _sc_spmm(m6, ed3)
    m7 = _combine_mm(p6, p["gnn7"], True)
    p7 = _sc_spmm(m7, ed3)
    m8 = _combine_mm(p7, p["gnn8"], True)
    p8 = _sc_spmm(m8, ed3)
    z_hat = _combine_relu(p8)

    adj_hat = _adj_hat(z_hat)
    q, q1, z_l = _q_kernel(pl_[0], z_ae, p["cluster"])
    return (x_bar, z_hat, adj_hat, z_ae, q, q1, z_l)


def _fusion_b_zi32(attp, g, e, w32):
    def body(att_ref, g_ref, e_ref, w_ref, out_ref):
        s = jnp.sum(att_ref[...], axis=0) / float(N)
        w0 = s[:, 0:1]
        w1 = s[:, 1:2]
        h = w0 * g_ref[...] + w1 * e_ref[...]
        out_ref[...] = _dot(h, w_ref[...])

    return pl.pallas_call(
        body,
        grid=(GRID,),
        in_specs=[_full_spec((GRID, 1, 128)), _row_spec(512), _row_spec(512),
                  _full_spec((512, 128))],
        out_specs=_row_spec(128),
        out_shape=jax.ShapeDtypeStruct((N, 128), F32),
    )(attp, g, e, w32)
